# baseline (speedup 1.0000x reference)
import functools

import jax
import jax.numpy as jnp
from jax import lax
from jax.experimental import pallas as pl
from jax.experimental.pallas import tpu as pltpu
from jax.experimental.pallas import tpu_sc as plsc

NC = 2
NS = 16
NW = NC * NS
L = 16
IN_DIMS = 4096
OUT_DIMS = 4096
ST = 1025
BLK = L * ST


def _sc_partials(x, W_vals, W_rows, W_cols):
    nnz = W_vals.shape[0]
    T = (nnz // (NW * L)) * L
    lft = nnz - NW * T
    nbf, tail = divmod(T, BLK)
    mesh = plsc.VectorSubcoreMesh(core_axis_name="c", subcore_axis_name="s")

    @functools.partial(
        pl.kernel,
        out_type=jax.ShapeDtypeStruct((2 * NW, OUT_DIMS), jnp.float32),
        mesh=mesh,
        compiler_params=pltpu.CompilerParams(needs_layout_passes=False),
        scratch_types=[
            pltpu.VMEM((IN_DIMS,), jnp.float32),
            pltpu.VMEM((BLK,), jnp.float32),
            pltpu.VMEM((BLK,), jnp.int32),
            pltpu.VMEM((BLK,), jnp.int32),
            pltpu.VMEM((BLK,), jnp.float32),
            pltpu.VMEM((BLK,), jnp.int32),
            pltpu.VMEM((BLK,), jnp.int32),
            pltpu.VMEM((OUT_DIMS,), jnp.float32),
            pltpu.VMEM((OUT_DIMS,), jnp.float32),
            pltpu.VMEM((NW * L,), jnp.float32),
            pltpu.VMEM((NW * L,), jnp.int32),
            pltpu.VMEM((NW * L,), jnp.int32),
            pltpu.SemaphoreType.DMA,
            pltpu.SemaphoreType.DMA,
            pltpu.SemaphoreType.DMA,
        ],
    )
    def body(x_hbm, vals_hbm, rows_hbm, cols_hbm, out_hbm,
             xv, valsv0, colsv0, rowsv0, valsv1, colsv1, rowsv1, accv, accbv,
             lvalsv, lcolsv, lrowsv, sem0, sem1, sem2):
        wid = lax.axis_index("s") * NC + lax.axis_index("c")
        slots = ((valsv0, colsv0, rowsv0, sem0), (valsv1, colsv1, rowsv1, sem1))
        base = wid * T
        iota = lax.iota(jnp.int32, L)

        def start_in(b, slot, size=BLK):
            valsv, colsv, rowsv, sem = slot
            off = base + b * BLK
            pltpu.async_copy(vals_hbm.at[pl.ds(off, size)], valsv.at[pl.ds(0, size)], sem)
            pltpu.async_copy(cols_hbm.at[pl.ds(off, size)], colsv.at[pl.ds(0, size)], sem)
            pltpu.async_copy(rows_hbm.at[pl.ds(off, size)], rowsv.at[pl.ds(0, size)], sem)

        def wait_in(slot, size=BLK):
            valsv, colsv, rowsv, sem = slot
            pltpu.make_async_copy(vals_hbm.at[pl.ds(0, size)], valsv.at[pl.ds(0, size)], sem).wait()
            pltpu.make_async_copy(cols_hbm.at[pl.ds(0, size)], colsv.at[pl.ds(0, size)], sem).wait()
            pltpu.make_async_copy(rows_hbm.at[pl.ds(0, size)], rowsv.at[pl.ds(0, size)], sem).wait()

        def strided_groups(slot, st):
            valsv, colsv, rowsv, _ = slot
            lane_base = iota * st

            def one(w, acc):
                idx = lane_base + w
                v16 = plsc.load_gather(valsv, [idx])
                c16 = plsc.load_gather(colsv, [idx])
                r16 = plsc.load_gather(rowsv, [idx])
                xg = plsc.load_gather(xv, [c16])
                plsc.addupdate_scatter(acc, [r16], v16 * xg)

            @plsc.parallel_loop(0, st - 1, 2, unroll=8)
            def grp(w):
                one(w, accv)
                one(w + 1, accbv)

            if st % 2 == 1:
                one(st - 1, accv)

        def cont_groups(slot, off0, n):
            valsv, colsv, rowsv, _ = slot

            def grp(j, _):
                sl = pl.ds(pl.multiple_of(off0 + j * L, L), L)
                xg = plsc.load_gather(xv, [colsv[sl]])
                plsc.addupdate_scatter(accv, [rowsv[sl]], valsv[sl] * xg)
                return 0
            lax.fori_loop(0, n, grp, 0)

        if nbf >= 1:
            start_in(0, slots[0])
        if nbf >= 2:
            start_in(1, slots[1])
        if lft:
            @pl.when(wid == NW - 1)
            def _lft_pre():
                off = NW * T
                pltpu.async_copy(vals_hbm.at[pl.ds(off, lft)], lvalsv.at[pl.ds(0, lft)], sem2)
                pltpu.async_copy(cols_hbm.at[pl.ds(off, lft)], lcolsv.at[pl.ds(0, lft)], sem2)
                pltpu.async_copy(rows_hbm.at[pl.ds(off, lft)], lrowsv.at[pl.ds(0, lft)], sem2)
        pltpu.sync_copy(x_hbm, xv)

        def zero(j, _):
            accv[pl.ds(pl.multiple_of(j * L, L), L)] = jnp.zeros((L,), jnp.float32)
            accbv[pl.ds(pl.multiple_of(j * L, L), L)] = jnp.zeros((L,), jnp.float32)
            return 0
        lax.fori_loop(0, OUT_DIMS // L, zero, 0)

        def blk_body(b, _):
            def do(s):
                wait_in(slots[s])
                strided_groups(slots[s], ST)

                @pl.when(b + 2 < nbf)
                def _():
                    start_in(b + 2, slots[s])

                if tail and nbf >= 2:
                    @pl.when(b + 2 == nbf)
                    def _():
                        start_in(nbf, slots[s], size=tail)

            @pl.when(b % 2 == 0)
            def _():
                do(0)

            @pl.when(b % 2 == 1)
            def _():
                do(1)
            return 0
        lax.fori_loop(0, nbf, blk_body, 0)

        if tail:
            q = tail // L
            st_t = q if q % 2 == 1 else q - 1
            tslot = slots[nbf % 2]
            if nbf >= 2:
                wait_in(tslot, size=tail)
            else:
                valsv, colsv, rowsv, _ = tslot
                off = base + nbf * BLK
                pltpu.sync_copy(vals_hbm.at[pl.ds(off, tail)], valsv.at[pl.ds(0, tail)])
                pltpu.sync_copy(cols_hbm.at[pl.ds(off, tail)], colsv.at[pl.ds(0, tail)])
                pltpu.sync_copy(rows_hbm.at[pl.ds(off, tail)], rowsv.at[pl.ds(0, tail)])
            if st_t >= 1:
                strided_groups(tslot, st_t)
            rest = tail - L * st_t
            if rest:
                cont_groups(tslot, L * st_t, rest // L)

        if lft:
            @pl.when(wid == NW - 1)
            def _lft():
                for _ in range(3):
                    pltpu.make_async_copy(
                        vals_hbm.at[pl.ds(0, lft)], lvalsv.at[pl.ds(0, lft)], sem2
                    ).wait()
                lslot = (lvalsv, lcolsv, lrowsv, sem2)
                nfull, rem = divmod(lft, L)
                cont_groups(lslot, 0, nfull)
                if rem:
                    sl = pl.ds(nfull * L, L)
                    m = iota < rem
                    c16 = jnp.where(m, lcolsv[sl], 0)
                    r16 = jnp.where(m, lrowsv[sl], 0)
                    v16 = jnp.where(m, lvalsv[sl], jnp.float32(0.0))
                    xg = plsc.load_gather(xv, [c16])
                    plsc.addupdate_scatter(accv, [r16], v16 * xg)

        pltpu.sync_copy(accv, out_hbm.at[2 * wid])
        pltpu.sync_copy(accbv, out_hbm.at[2 * wid + 1])

    return body(x, W_vals, W_rows, W_cols)


def _tc_reduce(partials, b):
    def body(p_ref, b_ref, o_ref):
        o_ref[...] = jnp.sum(p_ref[...], axis=0) + b_ref[...]
    return pl.pallas_call(
        body,
        out_shape=jax.ShapeDtypeStruct((OUT_DIMS,), jnp.float32),
    )(partials, b)


def kernel(x, W_vals, W_rows, W_cols, B):
    partials = _sc_partials(x, W_vals, W_rows, W_cols)
    return _tc_reduce(partials, B)


# speedup vs baseline: 1.0433x; 1.0433x over previous
import functools

import jax
import jax.numpy as jnp
from jax import lax
from jax.experimental import pallas as pl
from jax.experimental.pallas import tpu as pltpu
from jax.experimental.pallas import tpu_sc as plsc

NC = 2
NS = 16
NW = NC * NS
L = 16
IN_DIMS = 4096
OUT_DIMS = 4096
ST = 1025
BLK = L * ST


def _sc_partials(x, W_vals, W_rows, W_cols):
    nnz = W_vals.shape[0]
    T = (nnz // (NW * L)) * L
    lft = nnz - NW * T
    nbf, tail = divmod(T, BLK)
    mesh = plsc.VectorSubcoreMesh(core_axis_name="c", subcore_axis_name="s")

    @functools.partial(
        pl.kernel,
        out_type=jax.ShapeDtypeStruct((2 * NW, OUT_DIMS), jnp.float32),
        mesh=mesh,
        compiler_params=pltpu.CompilerParams(needs_layout_passes=False),
        scratch_types=[
            pltpu.VMEM((IN_DIMS,), jnp.float32),
            pltpu.VMEM((BLK,), jnp.float32),
            pltpu.VMEM((BLK,), jnp.int32),
            pltpu.VMEM((BLK,), jnp.int32),
            pltpu.VMEM((BLK,), jnp.float32),
            pltpu.VMEM((BLK,), jnp.int32),
            pltpu.VMEM((BLK,), jnp.int32),
            pltpu.VMEM((OUT_DIMS,), jnp.float32),
            pltpu.VMEM((OUT_DIMS,), jnp.float32),
            pltpu.VMEM((NW * L,), jnp.float32),
            pltpu.VMEM((NW * L,), jnp.int32),
            pltpu.VMEM((NW * L,), jnp.int32),
            pltpu.SemaphoreType.DMA,
            pltpu.SemaphoreType.DMA,
            pltpu.SemaphoreType.DMA,
            pltpu.SemaphoreType.DMA,
        ],
    )
    def body(x_hbm, vals_hbm, rows_hbm, cols_hbm, out_hbm,
             xv, valsv0, colsv0, rowsv0, valsv1, colsv1, rowsv1, accv, accbv,
             lvalsv, lcolsv, lrowsv, sem0, sem1, sem2, sem3):
        wid = lax.axis_index("s") * NC + lax.axis_index("c")
        slots = ((valsv0, colsv0, rowsv0, sem0), (valsv1, colsv1, rowsv1, sem1))
        base = wid * T
        iota = lax.iota(jnp.int32, L)

        def start_in(b, slot, size=BLK):
            valsv, colsv, rowsv, sem = slot
            off = base + b * BLK
            pltpu.async_copy(vals_hbm.at[pl.ds(off, size)], valsv.at[pl.ds(0, size)], sem)
            pltpu.async_copy(cols_hbm.at[pl.ds(off, size)], colsv.at[pl.ds(0, size)], sem)
            pltpu.async_copy(rows_hbm.at[pl.ds(off, size)], rowsv.at[pl.ds(0, size)], sem)

        def wait_in(slot, size=BLK):
            valsv, colsv, rowsv, sem = slot
            pltpu.make_async_copy(vals_hbm.at[pl.ds(0, size)], valsv.at[pl.ds(0, size)], sem).wait()
            pltpu.make_async_copy(cols_hbm.at[pl.ds(0, size)], colsv.at[pl.ds(0, size)], sem).wait()
            pltpu.make_async_copy(rows_hbm.at[pl.ds(0, size)], rowsv.at[pl.ds(0, size)], sem).wait()

        def strided_groups(slot, st):
            valsv, colsv, rowsv, _ = slot
            lane_base = iota * st

            def one(w, acc):
                idx = lane_base + w
                v16 = plsc.load_gather(valsv, [idx])
                c16 = plsc.load_gather(colsv, [idx])
                r16 = plsc.load_gather(rowsv, [idx])
                xg = plsc.load_gather(xv, [c16])
                plsc.addupdate_scatter(acc, [r16], v16 * xg)

            @plsc.parallel_loop(0, st - 1, 2, unroll=4)
            def grp(w):
                one(w, accv)
                one(w + 1, accbv)

            if st % 2 == 1:
                one(st - 1, accv)

        def cont_groups(slot, off0, n):
            valsv, colsv, rowsv, _ = slot

            def grp(j, _):
                sl = pl.ds(pl.multiple_of(off0 + j * L, L), L)
                xg = plsc.load_gather(xv, [colsv[sl]])
                plsc.addupdate_scatter(accv, [rowsv[sl]], valsv[sl] * xg)
                return 0
            lax.fori_loop(0, n, grp, 0)

        if nbf >= 1:
            start_in(0, slots[0])
        if nbf >= 2:
            start_in(1, slots[1])
        if lft:
            @pl.when(wid == NW - 1)
            def _lft_pre():
                off = NW * T
                pltpu.async_copy(vals_hbm.at[pl.ds(off, lft)], lvalsv.at[pl.ds(0, lft)], sem2)
                pltpu.async_copy(cols_hbm.at[pl.ds(off, lft)], lcolsv.at[pl.ds(0, lft)], sem2)
                pltpu.async_copy(rows_hbm.at[pl.ds(off, lft)], lrowsv.at[pl.ds(0, lft)], sem2)
        pltpu.async_copy(x_hbm, xv, sem3)

        z16 = jnp.zeros((L,), jnp.float32)

        @plsc.parallel_loop(0, OUT_DIMS // L, 1, unroll=8)
        def zero(j):
            accv[pl.ds(pl.multiple_of(j * L, L), L)] = z16
            accbv[pl.ds(pl.multiple_of(j * L, L), L)] = z16

        pltpu.make_async_copy(x_hbm, xv, sem3).wait()

        def blk_body(b, _):
            def do(s):
                wait_in(slots[s])
                strided_groups(slots[s], ST)

                @pl.when(b + 2 < nbf)
                def _():
                    start_in(b + 2, slots[s])

                if tail and nbf >= 2:
                    @pl.when(b + 2 == nbf)
                    def _():
                        start_in(nbf, slots[s], size=tail)

            @pl.when(b % 2 == 0)
            def _():
                do(0)

            @pl.when(b % 2 == 1)
            def _():
                do(1)
            return 0
        lax.fori_loop(0, nbf, blk_body, 0)

        if tail:
            q = tail // L
            st_t = q if q % 2 == 1 else q - 1
            tslot = slots[nbf % 2]
            if nbf >= 2:
                wait_in(tslot, size=tail)
            else:
                valsv, colsv, rowsv, _ = tslot
                off = base + nbf * BLK
                pltpu.sync_copy(vals_hbm.at[pl.ds(off, tail)], valsv.at[pl.ds(0, tail)])
                pltpu.sync_copy(cols_hbm.at[pl.ds(off, tail)], colsv.at[pl.ds(0, tail)])
                pltpu.sync_copy(rows_hbm.at[pl.ds(off, tail)], rowsv.at[pl.ds(0, tail)])
            if st_t >= 1:
                strided_groups(tslot, st_t)
            rest = tail - L * st_t
            if rest:
                cont_groups(tslot, L * st_t, rest // L)

        if lft:
            @pl.when(wid == NW - 1)
            def _lft():
                for _ in range(3):
                    pltpu.make_async_copy(
                        vals_hbm.at[pl.ds(0, lft)], lvalsv.at[pl.ds(0, lft)], sem2
                    ).wait()
                lslot = (lvalsv, lcolsv, lrowsv, sem2)
                nfull, rem = divmod(lft, L)
                cont_groups(lslot, 0, nfull)
                if rem:
                    sl = pl.ds(nfull * L, L)
                    m = iota < rem
                    c16 = jnp.where(m, lcolsv[sl], 0)
                    r16 = jnp.where(m, lrowsv[sl], 0)
                    v16 = jnp.where(m, lvalsv[sl], jnp.float32(0.0))
                    xg = plsc.load_gather(xv, [c16])
                    plsc.addupdate_scatter(accv, [r16], v16 * xg)

        pltpu.sync_copy(accv, out_hbm.at[2 * wid])
        pltpu.sync_copy(accbv, out_hbm.at[2 * wid + 1])

    return body(x, W_vals, W_rows, W_cols)


def _tc_reduce(partials, b):
    def body(p_ref, b_ref, o_ref):
        o_ref[...] = jnp.sum(p_ref[...], axis=0) + b_ref[...]
    return pl.pallas_call(
        body,
        out_shape=jax.ShapeDtypeStruct((OUT_DIMS,), jnp.float32),
    )(partials, b)


def kernel(x, W_vals, W_rows, W_cols, B):
    partials = _sc_partials(x, W_vals, W_rows, W_cols)
    return _tc_reduce(partials, B)


# speedup vs baseline: 1.0476x; 1.0040x over previous
import functools

import jax
import jax.numpy as jnp
from jax import lax
from jax.experimental import pallas as pl
from jax.experimental.pallas import tpu as pltpu
from jax.experimental.pallas import tpu_sc as plsc

NC = 2
NS = 16
NW = NC * NS
L = 16
IN_DIMS = 4096
OUT_DIMS = 4096
ST = 1025
BLK = L * ST


def _sc_partials(x, W_vals, W_rows, W_cols):
    nnz = W_vals.shape[0]
    T = (nnz // (NW * L)) * L
    lft = nnz - NW * T
    nbf, tail = divmod(T, BLK)
    mesh = plsc.VectorSubcoreMesh(core_axis_name="c", subcore_axis_name="s")

    @functools.partial(
        pl.kernel,
        out_type=jax.ShapeDtypeStruct((4 * NW, OUT_DIMS), jnp.float32),
        mesh=mesh,
        compiler_params=pltpu.CompilerParams(needs_layout_passes=False),
        scratch_types=[
            pltpu.VMEM((IN_DIMS,), jnp.float32),
            pltpu.VMEM((BLK,), jnp.float32),
            pltpu.VMEM((BLK,), jnp.int32),
            pltpu.VMEM((BLK,), jnp.int32),
            pltpu.VMEM((BLK,), jnp.float32),
            pltpu.VMEM((BLK,), jnp.int32),
            pltpu.VMEM((BLK,), jnp.int32),
            pltpu.VMEM((OUT_DIMS,), jnp.float32),
            pltpu.VMEM((OUT_DIMS,), jnp.float32),
            pltpu.VMEM((OUT_DIMS,), jnp.float32),
            pltpu.VMEM((OUT_DIMS,), jnp.float32),
            pltpu.VMEM((NW * L,), jnp.float32),
            pltpu.VMEM((NW * L,), jnp.int32),
            pltpu.VMEM((NW * L,), jnp.int32),
            pltpu.SemaphoreType.DMA,
            pltpu.SemaphoreType.DMA,
            pltpu.SemaphoreType.DMA,
            pltpu.SemaphoreType.DMA,
        ],
    )
    def body(x_hbm, vals_hbm, rows_hbm, cols_hbm, out_hbm,
             xv, valsv0, colsv0, rowsv0, valsv1, colsv1, rowsv1,
             accv, accbv, acccv, accdv,
             lvalsv, lcolsv, lrowsv, sem0, sem1, sem2, sem3):
        wid = lax.axis_index("s") * NC + lax.axis_index("c")
        slots = ((valsv0, colsv0, rowsv0, sem0), (valsv1, colsv1, rowsv1, sem1))
        base = wid * T
        iota = lax.iota(jnp.int32, L)

        def start_in(b, slot, size=BLK):
            valsv, colsv, rowsv, sem = slot
            off = base + b * BLK
            pltpu.async_copy(vals_hbm.at[pl.ds(off, size)], valsv.at[pl.ds(0, size)], sem)
            pltpu.async_copy(cols_hbm.at[pl.ds(off, size)], colsv.at[pl.ds(0, size)], sem)
            pltpu.async_copy(rows_hbm.at[pl.ds(off, size)], rowsv.at[pl.ds(0, size)], sem)

        def wait_in(slot, size=BLK):
            valsv, colsv, rowsv, sem = slot
            pltpu.make_async_copy(vals_hbm.at[pl.ds(0, size)], valsv.at[pl.ds(0, size)], sem).wait()
            pltpu.make_async_copy(cols_hbm.at[pl.ds(0, size)], colsv.at[pl.ds(0, size)], sem).wait()
            pltpu.make_async_copy(rows_hbm.at[pl.ds(0, size)], rowsv.at[pl.ds(0, size)], sem).wait()

        def strided_groups(slot, st):
            valsv, colsv, rowsv, _ = slot
            lane_base = iota * st

            def one(w, acc):
                idx = lane_base + w
                v16 = plsc.load_gather(valsv, [idx])
                c16 = plsc.load_gather(colsv, [idx])
                r16 = plsc.load_gather(rowsv, [idx])
                xg = plsc.load_gather(xv, [c16])
                plsc.addupdate_scatter(acc, [r16], v16 * xg)

            accs = (accv, accbv, acccv, accdv)
            m4 = (st // 4) * 4

            @plsc.parallel_loop(0, m4, 4, unroll=2)
            def grp(w):
                for k in range(4):
                    one(w + k, accs[k])

            for k in range(st - m4):
                one(m4 + k, accs[k])

        def cont_groups(slot, off0, n):
            valsv, colsv, rowsv, _ = slot

            def grp(j, _):
                sl = pl.ds(pl.multiple_of(off0 + j * L, L), L)
                xg = plsc.load_gather(xv, [colsv[sl]])
                plsc.addupdate_scatter(accv, [rowsv[sl]], valsv[sl] * xg)
                return 0
            lax.fori_loop(0, n, grp, 0)

        if nbf >= 1:
            start_in(0, slots[0])
        if nbf >= 2:
            start_in(1, slots[1])
        if lft:
            @pl.when(wid == NW - 1)
            def _lft_pre():
                off = NW * T
                pltpu.async_copy(vals_hbm.at[pl.ds(off, lft)], lvalsv.at[pl.ds(0, lft)], sem2)
                pltpu.async_copy(cols_hbm.at[pl.ds(off, lft)], lcolsv.at[pl.ds(0, lft)], sem2)
                pltpu.async_copy(rows_hbm.at[pl.ds(off, lft)], lrowsv.at[pl.ds(0, lft)], sem2)
        pltpu.async_copy(x_hbm, xv, sem3)

        z16 = jnp.zeros((L,), jnp.float32)

        @plsc.parallel_loop(0, OUT_DIMS // L, 1, unroll=8)
        def zero(j):
            accv[pl.ds(pl.multiple_of(j * L, L), L)] = z16
            accbv[pl.ds(pl.multiple_of(j * L, L), L)] = z16
            acccv[pl.ds(pl.multiple_of(j * L, L), L)] = z16
            accdv[pl.ds(pl.multiple_of(j * L, L), L)] = z16

        pltpu.make_async_copy(x_hbm, xv, sem3).wait()

        def blk_body(b, _):
            def do(s):
                wait_in(slots[s])
                strided_groups(slots[s], ST)

                @pl.when(b + 2 < nbf)
                def _():
                    start_in(b + 2, slots[s])

                if tail and nbf >= 2:
                    @pl.when(b + 2 == nbf)
                    def _():
                        start_in(nbf, slots[s], size=tail)

            @pl.when(b % 2 == 0)
            def _():
                do(0)

            @pl.when(b % 2 == 1)
            def _():
                do(1)
            return 0
        lax.fori_loop(0, nbf, blk_body, 0)

        if tail:
            q = tail // L
            st_t = q if q % 2 == 1 else q - 1
            tslot = slots[nbf % 2]
            if nbf >= 2:
                wait_in(tslot, size=tail)
            else:
                valsv, colsv, rowsv, _ = tslot
                off = base + nbf * BLK
                pltpu.sync_copy(vals_hbm.at[pl.ds(off, tail)], valsv.at[pl.ds(0, tail)])
                pltpu.sync_copy(cols_hbm.at[pl.ds(off, tail)], colsv.at[pl.ds(0, tail)])
                pltpu.sync_copy(rows_hbm.at[pl.ds(off, tail)], rowsv.at[pl.ds(0, tail)])
            if st_t >= 1:
                strided_groups(tslot, st_t)
            rest = tail - L * st_t
            if rest:
                cont_groups(tslot, L * st_t, rest // L)

        if lft:
            @pl.when(wid == NW - 1)
            def _lft():
                for _ in range(3):
                    pltpu.make_async_copy(
                        vals_hbm.at[pl.ds(0, lft)], lvalsv.at[pl.ds(0, lft)], sem2
                    ).wait()
                lslot = (lvalsv, lcolsv, lrowsv, sem2)
                nfull, rem = divmod(lft, L)
                cont_groups(lslot, 0, nfull)
                if rem:
                    sl = pl.ds(nfull * L, L)
                    m = iota < rem
                    c16 = jnp.where(m, lcolsv[sl], 0)
                    r16 = jnp.where(m, lrowsv[sl], 0)
                    v16 = jnp.where(m, lvalsv[sl], jnp.float32(0.0))
                    xg = plsc.load_gather(xv, [c16])
                    plsc.addupdate_scatter(accv, [r16], v16 * xg)

        pltpu.sync_copy(accv, out_hbm.at[4 * wid])
        pltpu.sync_copy(accbv, out_hbm.at[4 * wid + 1])
        pltpu.sync_copy(acccv, out_hbm.at[4 * wid + 2])
        pltpu.sync_copy(accdv, out_hbm.at[4 * wid + 3])

    return body(x, W_vals, W_rows, W_cols)


def _tc_reduce(partials, b):
    def body(p_ref, b_ref, o_ref):
        o_ref[...] = jnp.sum(p_ref[...], axis=0) + b_ref[...]
    return pl.pallas_call(
        body,
        out_shape=jax.ShapeDtypeStruct((OUT_DIMS,), jnp.float32),
    )(partials, b)


def kernel(x, W_vals, W_rows, W_cols, B):
    partials = _sc_partials(x, W_vals, W_rows, W_cols)
    return _tc_reduce(partials, B)
